# Initial kernel scaffold; baseline (speedup 1.0000x reference)
import jax
import jax.numpy as jnp
from jax.experimental import pallas as pl


def kernel(points, roots):
    raise NotImplementedError("write your pallas kernel here")


# speedup vs baseline: 3.2441x; 3.2441x over previous
import functools

import jax
import jax.numpy as jnp
from jax import lax
from jax.experimental import pallas as pl
from jax.experimental.pallas import tpu as pltpu

PATCH_K = 32
R_TILE = 256


def _dist_topk_body(roots_ref, points_ref, dist_out_ref, idx_out_ref,
                    kdist_out_ref, scratch_ref):
    rr = roots_ref[0]
    pp = points_ref[0]
    n = pp.shape[0]
    dot = lax.dot_general(rr, pp, (((1,), (1,)), ((), ())),
                          preferred_element_type=jnp.float32)
    r0 = jnp.sum(rr * rr, axis=1)
    r1 = jnp.sum(pp * pp, axis=1)
    d = r0[:, None] - 2.0 * dot + r1[None, :]
    dist_out_ref[0] = d
    scratch_ref[...] = d

    col = lax.broadcasted_iota(jnp.int32, (R_TILE, n), 1)
    ms = []
    idxs = []
    for _ in range(PATCH_K):
        dj = scratch_ref[...]
        m = jnp.min(dj, axis=1)
        a = jnp.min(jnp.where(dj == m[:, None], col, n), axis=1)
        ms.append(m)
        idxs.append(a)
        scratch_ref[...] = jnp.where(col == a[:, None], jnp.inf, dj)
    mstack = jnp.stack(ms, axis=1)
    astack = jnp.stack(idxs, axis=1)
    idx_out_ref[0] = astack
    kdist_out_ref[0] = jnp.sqrt(jnp.maximum(mstack, 1e-5))


@functools.partial(jax.jit, static_argnums=())
def _dist_topk(points, roots):
    B, N, _ = points.shape
    R = roots.shape[1]
    grid = (B, R // R_TILE)
    return pl.pallas_call(
        _dist_topk_body,
        grid=grid,
        in_specs=[
            pl.BlockSpec((1, R_TILE, 3), lambda b, r: (b, r, 0)),
            pl.BlockSpec((1, N, 3), lambda b, r: (b, 0, 0)),
        ],
        out_specs=[
            pl.BlockSpec((1, R_TILE, N), lambda b, r: (b, r, 0)),
            pl.BlockSpec((1, R_TILE, PATCH_K), lambda b, r: (b, r, 0)),
            pl.BlockSpec((1, R_TILE, PATCH_K), lambda b, r: (b, r, 0)),
        ],
        out_shape=[
            jax.ShapeDtypeStruct((B, R, N), jnp.float32),
            jax.ShapeDtypeStruct((B, R, PATCH_K), jnp.int32),
            jax.ShapeDtypeStruct((B, R, PATCH_K), jnp.float32),
        ],
        scratch_shapes=[pltpu.VMEM((R_TILE, N), jnp.float32)],
    )(roots, points)


def kernel(points, roots):
    B, N, _ = points.shape
    R = roots.shape[1]
    sq_distance_mat, knn_idx, patches_dist = _dist_topk(points, roots)

    batch_idx = jnp.broadcast_to(
        jnp.arange(B, dtype=knn_idx.dtype).reshape(B, 1, 1), (B, R, PATCH_K))
    patches_idx = jnp.stack([batch_idx, knn_idx], axis=-1)

    patches = jnp.take_along_axis(points[:, :, None, :],
                                  knn_idx.reshape(B, -1, 1, 1), axis=1)
    patches = patches.reshape(B, R, PATCH_K, 3) - roots[:, :, None, :]
    return (patches, patches_idx, patches_dist, sq_distance_mat)


# speedup vs baseline: 3.2457x; 1.0005x over previous
import functools

import jax
import jax.numpy as jnp
from jax import lax
from jax.experimental import pallas as pl
from jax.experimental.pallas import tpu as pltpu

PATCH_K = 32
R_TILE = 256


def _dist_topk_body(roots_ref, points_ref, dist_out_ref, idx_out_ref,
                    kdist_out_ref, scratch_ref):
    rr = roots_ref[0]
    pp = points_ref[0]
    n = pp.shape[0]
    dot = lax.dot_general(rr, pp, (((1,), (1,)), ((), ())),
                          preferred_element_type=jnp.float32)
    r0 = jnp.sum(rr * rr, axis=1)
    r1 = jnp.sum(pp * pp, axis=1)
    d = r0[:, None] - 2.0 * dot + r1[None, :]
    dist_out_ref[0] = d
    scratch_ref[...] = d

    col = lax.broadcasted_iota(jnp.int32, (R_TILE, n), 1)
    ms = []
    idxs = []
    for _ in range(PATCH_K):
        dj = scratch_ref[...]
        m = jnp.min(dj, axis=1)
        a = jnp.min(jnp.where(dj == m[:, None], col, n), axis=1)
        ms.append(m)
        idxs.append(a)
        scratch_ref[...] = jnp.where(col == a[:, None], jnp.inf, dj)
    mstack = jnp.stack(ms, axis=1)
    astack = jnp.stack(idxs, axis=1)
    idx_out_ref[0] = astack
    kdist_out_ref[0] = jnp.sqrt(jnp.maximum(mstack, 1e-5))


@functools.partial(jax.jit, static_argnums=())
def _dist_topk(points, roots):
    B, N, _ = points.shape
    R = roots.shape[1]
    grid = (B, R // R_TILE)
    return pl.pallas_call(
        _dist_topk_body,
        grid=grid,
        in_specs=[
            pl.BlockSpec((1, R_TILE, 3), lambda b, r: (b, r, 0)),
            pl.BlockSpec((1, N, 3), lambda b, r: (b, 0, 0)),
        ],
        out_specs=[
            pl.BlockSpec((1, R_TILE, N), lambda b, r: (b, r, 0)),
            pl.BlockSpec((1, R_TILE, PATCH_K), lambda b, r: (b, r, 0)),
            pl.BlockSpec((1, R_TILE, PATCH_K), lambda b, r: (b, r, 0)),
        ],
        out_shape=[
            jax.ShapeDtypeStruct((B, R, N), jnp.float32),
            jax.ShapeDtypeStruct((B, R, PATCH_K), jnp.int32),
            jax.ShapeDtypeStruct((B, R, PATCH_K), jnp.float32),
        ],
        scratch_shapes=[pltpu.VMEM((R_TILE, N), jnp.float32)],
        compiler_params=pltpu.CompilerParams(
            dimension_semantics=("parallel", "parallel")),
    )(roots, points)


def kernel(points, roots):
    B, N, _ = points.shape
    R = roots.shape[1]
    sq_distance_mat, knn_idx, patches_dist = _dist_topk(points, roots)

    batch_idx = jnp.broadcast_to(
        jnp.arange(B, dtype=knn_idx.dtype).reshape(B, 1, 1), (B, R, PATCH_K))
    patches_idx = jnp.stack([batch_idx, knn_idx], axis=-1)

    patches = jnp.take_along_axis(points[:, :, None, :],
                                  knn_idx.reshape(B, -1, 1, 1), axis=1)
    patches = patches.reshape(B, R, PATCH_K, 3) - roots[:, :, None, :]
    return (patches, patches_idx, patches_dist, sq_distance_mat)
